# Initial kernel scaffold; baseline (speedup 1.0000x reference)
import jax
import jax.numpy as jnp
from jax.experimental import pallas as pl


def kernel(x, table):
    raise NotImplementedError("write your pallas kernel here")


# speedup vs baseline: 1.5598x; 1.5598x over previous
import functools

import jax
import jax.numpy as jnp
from jax import lax
from jax.experimental import pallas as pl
from jax.experimental.pallas import tpu as pltpu
from jax.experimental.pallas import tpu_sc as plsc

N = 1000000
EMBED_DIM = 32
BATCH = 16384
FIELDS = 26

NC = 2
NS = 16
NW = NC * NS

TOTAL = BATCH * FIELDS
PER_W = TOTAL // NW
CHUNK = 128
NCHUNK = PER_W // CHUNK
G = 8
NGROUP = NCHUNK // G

assert PER_W * NW == TOTAL
assert CHUNK * NCHUNK == PER_W
assert G * NGROUP == NCHUNK


def _body(x_hbm, table_hbm, out_hbm, idx_v, rows_v, sem):
    c = lax.axis_index("c")
    s = lax.axis_index("s")
    wid = s * NC + c
    base = wid * PER_W
    pltpu.sync_copy(x_hbm.at[wid], idx_v)

    @pl.loop(0, NGROUP)
    def _group(g):
        cps = []
        for b in range(G):
            cps.append(
                pltpu.async_copy(
                    table_hbm.at[idx_v.at[g * G + b]],
                    rows_v.at[pl.ds(b * CHUNK, CHUNK)],
                    sem,
                )
            )
        for cp in cps:
            cp.wait()
        pltpu.sync_copy(
            rows_v, out_hbm.at[pl.ds(base + g * (G * CHUNK), G * CHUNK)]
        )


_mesh = plsc.VectorSubcoreMesh(
    core_axis_name="c", subcore_axis_name="s", num_cores=NC, num_subcores=NS
)

_gather = pl.kernel(
    _body,
    out_type=jax.ShapeDtypeStruct((TOTAL, EMBED_DIM), jnp.float32),
    mesh=_mesh,
    scratch_types=[
        pltpu.VMEM((NCHUNK, CHUNK), jnp.int32),
        pltpu.VMEM((G * CHUNK, EMBED_DIM), jnp.float32),
        pltpu.SemaphoreType.DMA,
    ],
    compiler_params=pltpu.CompilerParams(use_tc_tiling_on_sc=False),
)


@jax.jit
def kernel(x, table):
    idx = x.astype(jnp.int32).reshape(NW, NCHUNK, CHUNK)
    out = _gather(idx, table)
    return out.reshape(BATCH, FIELDS, EMBED_DIM)


# speedup vs baseline: 1.5723x; 1.0080x over previous
import functools

import jax
import jax.numpy as jnp
from jax import lax
from jax.experimental import pallas as pl
from jax.experimental.pallas import tpu as pltpu
from jax.experimental.pallas import tpu_sc as plsc

N = 1000000
EMBED_DIM = 32
BATCH = 16384
FIELDS = 26

NC = 2
NS = 16
NW = NC * NS

TOTAL = BATCH * FIELDS
PER_W = TOTAL // NW
CHUNK = 128
NCHUNK = PER_W // CHUNK
G = 8
NGROUP = NCHUNK // G

assert PER_W * NW == TOTAL
assert CHUNK * NCHUNK == PER_W
assert G * NGROUP == NCHUNK


GROWS = G * CHUNK


def _body(x_hbm, table_hbm, out_hbm, idx_v, rows0, rows1, gs0, gs1, ws0, ws1):
    c = lax.axis_index("c")
    s = lax.axis_index("s")
    wid = s * NC + c
    base = wid * PER_W
    pltpu.sync_copy(x_hbm.at[wid], idx_v)

    bufs = ((rows0, gs0, ws0), (rows1, gs1, ws1))

    def fire_gather(g):
        buf, gsem, _ = bufs[g % 2]
        return [
            pltpu.async_copy(
                table_hbm.at[idx_v.at[g * G + b]],
                buf.at[pl.ds(b * CHUNK, CHUNK)],
                gsem,
            )
            for b in range(G)
        ]

    gpend = {0: fire_gather(0), 1: fire_gather(1)}
    wpend = {}
    for g in range(NGROUP):
        buf, _, wsem = bufs[g % 2]
        for cp in gpend.pop(g):
            cp.wait()
        wpend[g] = pltpu.async_copy(
            buf, out_hbm.at[pl.ds(base + g * GROWS, GROWS)], wsem
        )
        if g + 2 < NGROUP:
            wpend.pop(g).wait()
            gpend[g + 2] = fire_gather(g + 2)
    for cp in wpend.values():
        cp.wait()


_mesh = plsc.VectorSubcoreMesh(
    core_axis_name="c", subcore_axis_name="s", num_cores=NC, num_subcores=NS
)

_gather = pl.kernel(
    _body,
    out_type=jax.ShapeDtypeStruct((TOTAL, EMBED_DIM), jnp.float32),
    mesh=_mesh,
    scratch_types=[
        pltpu.VMEM((NCHUNK, CHUNK), jnp.int32),
        pltpu.VMEM((GROWS, EMBED_DIM), jnp.float32),
        pltpu.VMEM((GROWS, EMBED_DIM), jnp.float32),
        pltpu.SemaphoreType.DMA,
        pltpu.SemaphoreType.DMA,
        pltpu.SemaphoreType.DMA,
        pltpu.SemaphoreType.DMA,
    ],
    compiler_params=pltpu.CompilerParams(use_tc_tiling_on_sc=False),
)


@jax.jit
def kernel(x, table):
    idx = x.astype(jnp.int32).reshape(NW, NCHUNK, CHUNK)
    out = _gather(idx, table)
    return out.reshape(BATCH, FIELDS, EMBED_DIM)
